# Initial kernel scaffold; baseline (speedup 1.0000x reference)
import jax
import jax.numpy as jnp
from jax.experimental import pallas as pl


def kernel(x, edge_index, edge_weigth, batch, W1, a1s, a1d, b1, g1, be1, W2, a2s, a2d, b2, g2, be2, W3, a3s, a3d, b3, g3, be3, fc1W, fc1b, fc2W, fc2b, fc3W, fc3b):
    raise NotImplementedError("write your pallas kernel here")


# speedup vs baseline: 34.8274x; 34.8274x over previous
import functools

import jax
import jax.numpy as jnp
from jax import lax
from jax.experimental import pallas as pl
from jax.experimental.pallas import tpu as pltpu
from jax.experimental.pallas import tpu_sc as plsc

NC = 2
NS = 16
NW = NC * NS
SUB = 128

f32 = jnp.float32
i32 = jnp.int32


def _leaky(x, slope):
    return jnp.where(x >= 0, x, x * slope)


def _esed(a2, h):
    return lax.dot_general(a2, h, (((1,), (1,)), ((), ())),
                           preferred_element_type=f32)


def _tc_first_body(x_ref, w_ref, a2_ref, h_ref, esed_ref):
    h = jnp.dot(x_ref[...], w_ref[...], preferred_element_type=f32)
    h_ref[...] = h
    esed_ref[...] = _esed(a2_ref[...], h)


def _tc_first(x, W, a2):
    n = x.shape[0]
    k = W.shape[1]
    return pl.pallas_call(
        _tc_first_body,
        out_shape=[
            jax.ShapeDtypeStruct((n, k), f32),
            jax.ShapeDtypeStruct((2, n), f32),
        ],
    )(x, W, a2)


def _tc_mid_body(n, p_ref, b_ref, g_ref, be_ref, w_ref, a2_ref,
                 h_ref, esed_ref):
    o = p_ref[0, :n, :] + p_ref[1, :n, :] + b_ref[...]
    mu = jnp.mean(o, axis=0, keepdims=True)
    d = o - mu
    var = jnp.mean(d * d, axis=0, keepdims=True)
    xb = d * lax.rsqrt(var + 1e-5) * g_ref[...] + be_ref[...]
    xb = _leaky(xb, 0.01)
    h = jnp.dot(xb, w_ref[...], preferred_element_type=f32)
    h_ref[...] = h
    esed_ref[...] = _esed(a2_ref[...], h)


def _tc_mid(partials, b, g, be, W, a2, n):
    k = W.shape[1]
    return pl.pallas_call(
        functools.partial(_tc_mid_body, n),
        out_shape=[
            jax.ShapeDtypeStruct((n, k), f32),
            jax.ShapeDtypeStruct((2, n), f32),
        ],
    )(partials, b, g, be, W, a2)


def _tc_final_body(n, g_graphs, p_ref, b_ref, g_ref, be_ref, batch_ref,
                   fc1w_ref, fc1b_ref, fc2w_ref, fc2b_ref, fc3w_ref,
                   fc3b_ref, out_ref):
    o = p_ref[0, :n, :] + p_ref[1, :n, :] + b_ref[...]
    mu = jnp.mean(o, axis=0, keepdims=True)
    d = o - mu
    var = jnp.mean(d * d, axis=0, keepdims=True)
    xb = d * lax.rsqrt(var + 1e-5) * g_ref[...] + be_ref[...]
    h = _leaky(xb, 0.01)
    gid = lax.broadcasted_iota(i32, (g_graphs, n), 0)
    onehot = jnp.where(gid == batch_ref[...], 1.0, 0.0).astype(f32)
    pooled = jnp.dot(onehot, h, preferred_element_type=f32)
    o1 = _leaky(jnp.dot(pooled, fc1w_ref[...],
                        preferred_element_type=f32) + fc1b_ref[...], 0.01)
    o2 = _leaky(jnp.dot(o1, fc2w_ref[...],
                        preferred_element_type=f32) + fc2b_ref[...], 0.01)
    o3 = _leaky(jnp.dot(o2, fc3w_ref[...],
                        preferred_element_type=f32) + fc3b_ref[...], 0.01)
    out_ref[...] = o3


def _tc_final(partials, b, g, be, batch2d, fc1W, fc1b, fc2W, fc2b, fc3W,
              fc3b, n, g_graphs):
    return pl.pallas_call(
        functools.partial(_tc_final_body, n, g_graphs),
        out_shape=jax.ShapeDtypeStruct((g_graphs, fc3W.shape[1]), f32),
    )(partials, b, g, be, batch2d, fc1W, fc1b, fc2W, fc2b, fc3W, fc3b)


def _mesh():
    return plsc.VectorSubcoreMesh(
        core_axis_name="c", subcore_axis_name="s",
        num_cores=NC, num_subcores=NS)


def _pass_a(esed, idx, z1, n, n_pad, nch):
    sl = n_pad // NS

    @functools.partial(
        pl.kernel,
        out_type=[
            jax.ShapeDtypeStruct((NW, nch, SUB), f32),
            jax.ShapeDtypeStruct((NC, n_pad), f32),
        ],
        mesh=_mesh(),
        compiler_params=pltpu.CompilerParams(
            needs_layout_passes=False, use_tc_tiling_on_sc=False),
        scratch_types=[
            pltpu.VMEM((n,), f32),
            pltpu.VMEM((n,), f32),
            pltpu.VMEM((nch, 2, SUB), i32),
            pltpu.VMEM((nch, SUB), f32),
            pltpu.VMEM_SHARED((n_pad,), f32),
        ],
    )
    def body(esed_hbm, idx_hbm, z1_hbm, ex_hbm, s_hbm,
             es_v, ed_v, idx_v, ex_v, s_sh):
        c = lax.axis_index("c")
        sid = lax.axis_index("s")
        wid = sid * NC + c
        pltpu.sync_copy(z1_hbm.at[pl.ds(sid * sl, sl)],
                        s_sh.at[pl.ds(sid * sl, sl)])
        pltpu.sync_copy(esed_hbm.at[0], es_v)
        pltpu.sync_copy(esed_hbm.at[1], ed_v)
        pltpu.sync_copy(idx_hbm.at[wid], idx_v)
        plsc.subcore_barrier()

        def chunk(ch, carry):
            for j in range(SUB // 16):
                s_idx = idx_v[ch, 0, pl.ds(j * 16, 16)]
                d_idx = idx_v[ch, 1, pl.ds(j * 16, 16)]
                e = (plsc.load_gather(es_v, [s_idx])
                     + plsc.load_gather(ed_v, [d_idx]))
                e = jnp.maximum(e, e * 0.2)
                ex_v[ch, pl.ds(j * 16, 16)] = jnp.exp(e)
            pltpu.sync_copy(ex_v.at[ch], s_sh.at[idx_v.at[ch].at[1]],
                            add=True)
            return carry

        lax.fori_loop(0, nch, chunk, 0)
        pltpu.sync_copy(ex_v, ex_hbm.at[wid])
        plsc.subcore_barrier()
        pltpu.sync_copy(s_sh.at[pl.ds(sid * sl, sl)],
                        s_hbm.at[c].at[pl.ds(sid * sl, sl)])

    return body(esed, idx, z1)


_CBLK = 7


def _pass_c(h, s_partial, idx, ex, zk, n, n_pad, nch):
    k = h.shape[1]
    sl = n_pad // NS
    nblk = nch // _CBLK

    @functools.partial(
        pl.kernel,
        out_type=jax.ShapeDtypeStruct((NC, n_pad, k), f32),
        mesh=_mesh(),
        compiler_params=pltpu.CompilerParams(
            needs_layout_passes=False, use_tc_tiling_on_sc=False),
        scratch_types=[
            pltpu.VMEM((n_pad,), f32),
            pltpu.VMEM((n_pad,), f32),
            pltpu.VMEM((_CBLK, 2, SUB), i32),
            pltpu.VMEM((_CBLK, SUB), f32),
            pltpu.VMEM((SUB, k), f32),
            pltpu.VMEM((SUB,), f32),
            pltpu.VMEM_SHARED((n_pad, k), f32),
            pltpu.SemaphoreType.DMA,
        ],
    )
    def body(h_hbm, s_hbm, idx_hbm, ex_hbm, zk_hbm, out_hbm,
             rs_v, tmp_v, idx_v, exb_v, rows_v, alpha_v, out_sh, sem):
        c = lax.axis_index("c")
        sid = lax.axis_index("s")
        wid = sid * NC + c
        pltpu.sync_copy(zk_hbm.at[pl.ds(sid * sl, sl)],
                        out_sh.at[pl.ds(sid * sl, sl)])
        pltpu.sync_copy(s_hbm.at[0], rs_v)
        pltpu.sync_copy(s_hbm.at[1], tmp_v)

        def rsloop(i, carry):
            s16 = pl.ds(i * 16, 16)
            rs_v[s16] = 1.0 / (rs_v[s16] + tmp_v[s16] + 1e-16)
            return carry

        lax.fori_loop(0, n_pad // 16, rsloop, 0)
        plsc.subcore_barrier()

        def blk(b, carry):
            pltpu.sync_copy(idx_hbm.at[wid].at[pl.ds(b * _CBLK, _CBLK)],
                            idx_v)
            pltpu.sync_copy(ex_hbm.at[wid].at[pl.ds(b * _CBLK, _CBLK)],
                            exb_v)
            for j in range(_CBLK):
                pltpu.async_copy(
                    h_hbm.at[idx_v.at[j].at[0]], rows_v, sem).wait()
                for jj in range(SUB // 16):
                    d_idx = idx_v[j, 1, pl.ds(jj * 16, 16)]
                    rsg = plsc.load_gather(rs_v, [d_idx])
                    alpha_v[pl.ds(jj * 16, 16)] = (
                        exb_v[j, pl.ds(jj * 16, 16)] * rsg)

                def scale(i, carry2):
                    asp = plsc.load_gather(
                        alpha_v, [lax.broadcast(i, (16,))])
                    for kk in range(k // 16):
                        s16 = pl.ds(kk * 16, 16)
                        rows_v[i, s16] = rows_v[i, s16] * asp
                    return carry2

                lax.fori_loop(0, SUB, scale, 0)
                pltpu.sync_copy(rows_v, out_sh.at[idx_v.at[j].at[1]],
                                add=True)
            return carry

        lax.fori_loop(0, nblk, blk, 0)
        plsc.subcore_barrier()
        pltpu.sync_copy(out_sh.at[pl.ds(sid * sl, sl)],
                        out_hbm.at[c].at[pl.ds(sid * sl, sl)])

    return body(h, s_partial, idx, ex, zk)


def kernel(x, edge_index, edge_weigth, batch, W1, a1s, a1d, b1, g1, be1,
           W2, a2s, a2d, b2, g2, be2, W3, a3s, a3d, b3, g3, be3,
           fc1W, fc1b, fc2W, fc2b, fc3W, fc3b):
    n = x.shape[0]
    e = edge_index.shape[1]
    g_graphs = 64

    e_real = e + n
    nch = -(-e_real // (NW * SUB))
    nch = -(-nch // _CBLK) * _CBLK
    e_pad = NW * nch * SUB
    pad = e_pad - e_real
    n_pad = -(-n // (NS * 128)) * NS * 128

    loops = jnp.arange(n, dtype=i32)
    pad_src = jnp.arange(pad, dtype=i32) % n
    pad_dst = n + jnp.arange(pad, dtype=i32) % (n_pad - n)
    src = jnp.concatenate([edge_index[0], loops, pad_src]).reshape(
        NW, nch, 1, SUB)
    dst = jnp.concatenate([edge_index[1], loops, pad_dst]).reshape(
        NW, nch, 1, SUB)
    idx = jnp.concatenate([src, dst], axis=2)

    z1 = jnp.zeros((n_pad,), f32)

    def layer(h_esed):
        h, esed = h_esed
        k = h.shape[1]
        ex, sp = _pass_a(esed, idx, z1, n, n_pad, nch)
        zk = jnp.zeros((n_pad, k), f32)
        return _pass_c(h, sp, idx, ex, zk, n, n_pad, nch)

    r2 = lambda v: v.reshape(1, -1)
    stk = lambda u, v: jnp.stack([u, v], axis=0)

    partials = layer(_tc_first(x, W1, stk(a1s, a1d)))
    partials = layer(
        _tc_mid(partials, r2(b1), r2(g1), r2(be1), W2, stk(a2s, a2d), n))
    partials = layer(
        _tc_mid(partials, r2(b2), r2(g2), r2(be2), W3, stk(a3s, a3d), n))

    return _tc_final(partials, r2(b3), r2(g3), r2(be3),
                     batch.reshape(1, n).astype(i32),
                     fc1W, r2(fc1b), fc2W, r2(fc2b), fc3W, r2(fc3b),
                     n, g_graphs)


# speedup vs baseline: 42.9969x; 1.2346x over previous
import functools

import jax
import jax.numpy as jnp
from jax import lax
from jax.experimental import pallas as pl
from jax.experimental.pallas import tpu as pltpu
from jax.experimental.pallas import tpu_sc as plsc

NC = 2
NS = 16
NW = NC * NS
SUB = 128
CBLK = 7

f32 = jnp.float32
i32 = jnp.int32


def _leaky(x, slope):
    return jnp.where(x >= 0, x, x * slope)


def _esed(a2, h):
    return lax.dot_general(a2, h, (((1,), (1,)), ((), ())),
                           preferred_element_type=f32,
                           precision=lax.Precision.HIGHEST)


def _tc_first_body(x_ref, w_ref, a2_ref, h_ref, esed_ref):
    h = jnp.dot(x_ref[...], w_ref[...], preferred_element_type=f32)
    h_ref[...] = h
    esed_ref[...] = _esed(a2_ref[...], h)


def _tc_first(x, W, a2):
    n = x.shape[0]
    k = W.shape[1]
    return pl.pallas_call(
        _tc_first_body,
        out_shape=[
            jax.ShapeDtypeStruct((n, k), f32),
            jax.ShapeDtypeStruct((2, n), f32),
        ],
    )(x, W, a2)


def _gat_epilogue(p_ref, s_ref, b_ref, g_ref, be_ref, n, col_split):
    if col_split:
        o = jnp.concatenate([p_ref[0, :n, :], p_ref[1, :n, :]], axis=1)
        s = s_ref[0, :n]
    else:
        o = p_ref[0, :n, :] + p_ref[1, :n, :]
        s = s_ref[0, :n] + s_ref[1, :n]
    rs = 1.0 / (s + 1e-16)
    o = o * rs[:, None] + b_ref[...]
    mu = jnp.mean(o, axis=0, keepdims=True)
    d = o - mu
    var = jnp.mean(d * d, axis=0, keepdims=True)
    xb = d * lax.rsqrt(var + 1e-5) * g_ref[...] + be_ref[...]
    return _leaky(xb, 0.01)


def _tc_mid_body(n, split_out, p_ref, s_ref, b_ref, g_ref, be_ref, w_ref,
                 a2_ref, h_ref, esed_ref):
    xb = _gat_epilogue(p_ref, s_ref, b_ref, g_ref, be_ref, n, False)
    h = jnp.dot(xb, w_ref[...], preferred_element_type=f32)
    if split_out:
        k2 = h.shape[1] // 2
        h_ref[0, :, :] = h[:, :k2]
        h_ref[1, :, :] = h[:, k2:]
    else:
        h_ref[...] = h
    esed_ref[...] = _esed(a2_ref[...], h)


def _tc_mid(partials, s_part, b, g, be, W, a2, n, split_out):
    k = W.shape[1]
    h_shape = (NC, n, k // 2) if split_out else (n, k)
    return pl.pallas_call(
        functools.partial(_tc_mid_body, n, split_out),
        out_shape=[
            jax.ShapeDtypeStruct(h_shape, f32),
            jax.ShapeDtypeStruct((2, n), f32),
        ],
    )(partials, s_part, b, g, be, W, a2)


def _tc_final_body(n, g_graphs, p_ref, s_ref, b_ref, g_ref, be_ref,
                   batch_ref, fc1w_ref, fc1b_ref, fc2w_ref, fc2b_ref,
                   fc3w_ref, fc3b_ref, out_ref):
    h = _gat_epilogue(p_ref, s_ref, b_ref, g_ref, be_ref, n, True)
    gid = lax.broadcasted_iota(i32, (g_graphs, n), 0)
    onehot = jnp.where(gid == batch_ref[...], 1.0, 0.0).astype(f32)
    hp = lax.Precision.HIGHEST
    pooled = jnp.dot(onehot, h, preferred_element_type=f32, precision=hp)
    o1 = _leaky(jnp.dot(pooled, fc1w_ref[...],
                        preferred_element_type=f32) + fc1b_ref[...], 0.01)
    o2 = _leaky(jnp.dot(o1, fc2w_ref[...],
                        preferred_element_type=f32) + fc2b_ref[...], 0.01)
    o3 = _leaky(jnp.dot(o2, fc3w_ref[...],
                        preferred_element_type=f32) + fc3b_ref[...], 0.01)
    out_ref[...] = o3


def _tc_final(partials, s_part, b, g, be, batch2d, fc1W, fc1b, fc2W, fc2b,
              fc3W, fc3b, n, g_graphs):
    return pl.pallas_call(
        functools.partial(_tc_final_body, n, g_graphs),
        out_shape=jax.ShapeDtypeStruct((g_graphs, fc3W.shape[1]), f32),
    )(partials, s_part, b, g, be, batch2d, fc1W, fc1b, fc2W, fc2b, fc3W,
      fc3b)


def _edge_pass(h, esed, idx, z1, zk, n, n_pad, nch, col_split):
    k = h.shape[-1]
    sl = n_pad // NS
    nblk = nch // CBLK
    nbuf = 1 if (not col_split and k == 128) else 2

    @functools.partial(
        pl.kernel,
        out_type=[
            jax.ShapeDtypeStruct((NC, n_pad), f32),
            jax.ShapeDtypeStruct((NC, n_pad, k), f32),
        ],
        mesh=_mesh(),
        compiler_params=pltpu.CompilerParams(
            needs_layout_passes=False, use_tc_tiling_on_sc=False),
        scratch_types=[
            pltpu.VMEM((n,), f32),
            pltpu.VMEM((n,), f32),
            pltpu.VMEM((CBLK, 2, SUB), i32),
            pltpu.VMEM((SUB,), f32),
            pltpu.VMEM((nbuf, SUB, k), f32),
            pltpu.VMEM_SHARED((n_pad,), f32),
            pltpu.VMEM_SHARED((n_pad, k), f32),
            pltpu.SemaphoreType.DMA,
            pltpu.SemaphoreType.DMA,
        ],
    )
    def body(h_hbm, esed_hbm, idx_hbm, z1_hbm, zk_hbm, s_hbm, out_hbm,
             es_v, ed_v, idx_v, ex_v, rows_v, s_sh, out_sh, sem0, sem1):
        c = lax.axis_index("c")
        sid = lax.axis_index("s")
        if col_split:
            h_op = h_hbm.at[c]
            idx_tile = idx_hbm.at[sid]
        else:
            h_op = h_hbm
            idx_tile = idx_hbm.at[sid * NC + c]
        sems = [sem0, sem1]

        pltpu.sync_copy(z1_hbm.at[pl.ds(sid * sl, sl)],
                        s_sh.at[pl.ds(sid * sl, sl)])
        pltpu.sync_copy(zk_hbm.at[pl.ds(sid * sl, sl)],
                        out_sh.at[pl.ds(sid * sl, sl)])
        pltpu.sync_copy(esed_hbm.at[0], es_v)
        pltpu.sync_copy(esed_hbm.at[1], ed_v)
        plsc.subcore_barrier()

        def blk(b, carry):
            pltpu.sync_copy(idx_tile.at[pl.ds(b * CBLK, CBLK)], idx_v)
            descs = [None, None]
            if nbuf == 2:
                descs[0] = pltpu.async_copy(
                    h_op.at[idx_v.at[0].at[0]], rows_v.at[0], sems[0])
            for j in range(CBLK):
                buf = j % nbuf
                if nbuf == 2:
                    descs[buf].wait()
                    if j + 1 < CBLK:
                        descs[1 - buf] = pltpu.async_copy(
                            h_op.at[idx_v.at[j + 1].at[0]],
                            rows_v.at[1 - buf], sems[1 - buf])
                else:
                    pltpu.async_copy(
                        h_op.at[idx_v.at[j].at[0]], rows_v.at[buf],
                        sems[buf]).wait()
                for jj in range(SUB // 16):
                    s16 = pl.ds(jj * 16, 16)
                    e = (plsc.load_gather(es_v, [idx_v[j, 0, s16]])
                         + plsc.load_gather(ed_v, [idx_v[j, 1, s16]]))
                    e = jnp.maximum(e, e * 0.2)
                    ex_v[s16] = jnp.exp(e)
                pltpu.sync_copy(ex_v, s_sh.at[idx_v.at[j].at[1]],
                                add=True)

                def scale(i, carry2):
                    asp = plsc.load_gather(
                        ex_v, [lax.broadcast(i, (16,))])
                    for kk in range(k // 16):
                        c16 = pl.ds(kk * 16, 16)
                        rows_v[buf, i, c16] = rows_v[buf, i, c16] * asp
                    return carry2

                lax.fori_loop(0, SUB, scale, 0)
                pltpu.sync_copy(rows_v.at[buf],
                                out_sh.at[idx_v.at[j].at[1]], add=True)
            return carry

        lax.fori_loop(0, nblk, blk, 0)
        plsc.subcore_barrier()
        pltpu.sync_copy(s_sh.at[pl.ds(sid * sl, sl)],
                        s_hbm.at[c].at[pl.ds(sid * sl, sl)])
        pltpu.sync_copy(out_sh.at[pl.ds(sid * sl, sl)],
                        out_hbm.at[c].at[pl.ds(sid * sl, sl)])

    return body(h, esed, idx, z1, zk)


def _mesh():
    return plsc.VectorSubcoreMesh(
        core_axis_name="c", subcore_axis_name="s",
        num_cores=NC, num_subcores=NS)


def kernel(x, edge_index, edge_weigth, batch, W1, a1s, a1d, b1, g1, be1,
           W2, a2s, a2d, b2, g2, be2, W3, a3s, a3d, b3, g3, be3,
           fc1W, fc1b, fc2W, fc2b, fc3W, fc3b):
    n = x.shape[0]
    e = edge_index.shape[1]
    g_graphs = 64

    e_real = e + n
    nch32 = -(-e_real // (NW * SUB))
    nch32 = -(-nch32 // CBLK) * CBLK
    nch16 = 2 * nch32
    e_pad = NW * nch32 * SUB
    pad = e_pad - e_real
    n_pad = -(-n // (NS * 128)) * NS * 128

    loops = jnp.arange(n, dtype=i32)
    pad_src = jnp.arange(pad, dtype=i32) % n
    pad_dst = n + jnp.arange(pad, dtype=i32) % (n_pad - n)
    src = jnp.concatenate([edge_index[0], loops, pad_src])
    dst = jnp.concatenate([edge_index[1], loops, pad_dst])
    idx32 = jnp.stack([src.reshape(NW, nch32, SUB),
                       dst.reshape(NW, nch32, SUB)], axis=2)
    idx16 = jnp.stack([src.reshape(NS, nch16, SUB),
                       dst.reshape(NS, nch16, SUB)], axis=2)

    z1 = jnp.zeros((n_pad,), f32)

    def layer(h_esed, col_split):
        h, esed = h_esed
        k = h.shape[-1]
        zk = jnp.zeros((n_pad, k), f32)
        return _edge_pass(h, esed, idx16 if col_split else idx32, z1, zk,
                          n, n_pad, nch16 if col_split else nch32,
                          col_split)

    r2 = lambda v: v.reshape(1, -1)
    stk = lambda u, v: jnp.stack([u, v], axis=0)

    sp, op = layer(_tc_first(x, W1, stk(a1s, a1d)), False)
    sp, op = layer(
        _tc_mid(op, sp, r2(b1), r2(g1), r2(be1), W2, stk(a2s, a2d), n,
                split_out=False), False)
    sp, op = layer(
        _tc_mid(op, sp, r2(b2), r2(g2), r2(be2), W3, stk(a3s, a3d), n,
                split_out=True), True)

    return _tc_final(op, sp, r2(b3), r2(g3), r2(be3),
                     batch.reshape(1, n).astype(i32),
                     fc1W, r2(fc1b), fc2W, r2(fc2b), fc3W, r2(fc3b),
                     n, g_graphs)
